# baseline (speedup 1.0000x reference)
import functools

import jax
import jax.numpy as jnp
from jax import lax
from jax.experimental import pallas as pl
from jax.experimental.pallas import tpu as pltpu
from jax.experimental.pallas import tpu_sc as plsc

N = 32
IN_CH = 512
HIDDEN = 256
HEADS = 4
OUT_CH = HIDDEN // HEADS
SPATIAL = 56 * 56
NUM_AU = 32
ROWS = N * IN_CH
LANES = 16

_info = plsc.get_sparse_core_info()
_NC = _info.num_cores
_NS = _info.num_subcores
NW = _NC * _NS
ROWS_PER_W = ROWS // NW
ROWS_PER_CHUNK = 8
CHUNK_ELEMS = ROWS_PER_CHUNK * SPATIAL
CHUNKS_PER_W = ROWS_PER_W // ROWS_PER_CHUNK
VECS_PER_ROW = SPATIAL // LANES


def _row_sums(buf_ref, out_ref):
    for r in range(ROWS_PER_CHUNK):
        base = r * SPATIAL
        acc = [jnp.zeros((LANES,), jnp.float32) for _ in range(4)]
        for j in range(VECS_PER_ROW):
            acc[j % 4] = acc[j % 4] + buf_ref[pl.ds(base + j * LANES, LANES)]
        out_ref[pl.ds(r * LANES, LANES)] = (acc[0] + acc[1]) + (acc[2] + acc[3])


def _sc_pool_body(x_ref, out_ref, buf_a, buf_b, row_out, sem_a, sem_b):
    wid = lax.axis_index("s") * _NC + lax.axis_index("c")
    w_elems = wid * ROWS_PER_W * SPATIAL
    w_out = wid * ROWS_PER_W * LANES

    pltpu.async_copy(x_ref.at[pl.ds(w_elems, CHUNK_ELEMS)], buf_a, sem_a)

    def step(t, carry):
        off = w_elems + (2 * t) * CHUNK_ELEMS

        pltpu.async_copy(x_ref.at[pl.ds(off + CHUNK_ELEMS, CHUNK_ELEMS)],
                         buf_b, sem_b)
        pltpu.make_async_copy(x_ref.at[pl.ds(off, CHUNK_ELEMS)],
                              buf_a, sem_a).wait()
        _row_sums(buf_a, row_out)
        pltpu.sync_copy(
            row_out,
            out_ref.at[pl.ds(w_out + (2 * t) * ROWS_PER_CHUNK * LANES,
                             ROWS_PER_CHUNK * LANES)])

        @pl.when(t < CHUNKS_PER_W // 2 - 1)
        def _prefetch_a():
            pltpu.async_copy(
                x_ref.at[pl.ds(off + 2 * CHUNK_ELEMS, CHUNK_ELEMS)],
                buf_a, sem_a)

        pltpu.make_async_copy(x_ref.at[pl.ds(off + CHUNK_ELEMS, CHUNK_ELEMS)],
                              buf_b, sem_b).wait()
        _row_sums(buf_b, row_out)
        pltpu.sync_copy(
            row_out,
            out_ref.at[pl.ds(w_out + (2 * t + 1) * ROWS_PER_CHUNK * LANES,
                             ROWS_PER_CHUNK * LANES)])
        return carry

    lax.fori_loop(0, CHUNKS_PER_W // 2, step, 0)


def _sc_pool(x_flat):
    mesh = plsc.VectorSubcoreMesh(core_axis_name="c", subcore_axis_name="s")
    kern = functools.partial(
        pl.kernel,
        mesh=mesh,
        out_type=jax.ShapeDtypeStruct((ROWS * LANES,), jnp.float32),
        scratch_types=[
            pltpu.VMEM((CHUNK_ELEMS,), jnp.float32),
            pltpu.VMEM((CHUNK_ELEMS,), jnp.float32),
            pltpu.VMEM((ROWS_PER_CHUNK * LANES,), jnp.float32),
            pltpu.SemaphoreType.DMA,
            pltpu.SemaphoreType.DMA,
        ],
    )(_sc_pool_body)
    return kern(x_flat)


def _tc_epilogue_body(p_ref, eiT_ref, wlin_ref, att_ref, bias_ref, wcls_ref,
                      bcls_ref, out_ref):
    sums = jnp.sum(p_ref[...], axis=2)
    xm = sums * (1.0 / SPATIAL)
    h = jnp.dot(xm, wlin_ref[...],
                preferred_element_type=jnp.float32)
    hh = h.reshape(N, HEADS, OUT_CH)
    att_src = att_ref[0:HEADS, :]
    att_dst = att_ref[HEADS:2 * HEADS, :]
    a_src = jnp.sum(hh * att_src[None, :, :], axis=-1)
    a_dst = jnp.sum(hh * att_dst[None, :, :], axis=-1)

    e = a_src[:, None, :] + a_dst[None, :, :]
    e = jnp.where(e >= 0, e, 0.2 * e)

    src = eiT_ref[:, 0:1]
    dst = eiT_ref[:, 1:2]
    num_edges = eiT_ref.shape[0]
    ids = lax.broadcasted_iota(jnp.int32, (num_edges, N), 1)
    oh_s = (src == ids).astype(jnp.float32)
    oh_d = (dst == ids).astype(jnp.float32)
    cmat = lax.dot_general(oh_s, oh_d, (((0,), (0,)), ((), ())),
                           preferred_element_type=jnp.float32)
    present = (cmat > 0.0).astype(jnp.float32)

    e_m = e + (present[:, :, None] - 1.0) * jnp.float32(1e30)
    mx = jnp.max(e_m, axis=0)
    ee = jnp.exp(jnp.minimum(e - mx[None, :, :], 0.0))
    ee = ee * cmat[:, :, None]
    denom = jnp.sum(ee, axis=0) + jnp.float32(1e-16)

    outs = []
    for hd in range(HEADS):
        w = ee[:, :, hd]
        num = lax.dot_general(w, h[:, hd * OUT_CH:(hd + 1) * OUT_CH],
                              (((0,), (0,)), ((), ())),
                              preferred_element_type=jnp.float32)
        outs.append(num / denom[:, hd][:, None])
    gat = jnp.concatenate(outs, axis=1) + bias_ref[...]
    act = gat * jax.nn.sigmoid(gat)
    logit = jnp.dot(act, wcls_ref[...],
                    preferred_element_type=jnp.float32) + bcls_ref[...]
    out_ref[...] = logit


def kernel(roi_feats, edge_index, W_lin, att_src, att_dst, bias_gat,
           W_cls, b_cls):
    x_flat = roi_feats.reshape(ROWS * SPATIAL)
    pooled16 = _sc_pool(x_flat).reshape(N, IN_CH, LANES)

    eiT = edge_index.T
    att = jnp.concatenate([att_src, att_dst], axis=0)
    bias2d = bias_gat.reshape(1, HIDDEN)
    bcls2d = b_cls.reshape(1, NUM_AU)

    out = pl.pallas_call(
        _tc_epilogue_body,
        grid=(1,),
        in_specs=[
            pl.BlockSpec(pooled16.shape, lambda g: (0, 0, 0)),
            pl.BlockSpec(eiT.shape, lambda g: (0, 0)),
            pl.BlockSpec(W_lin.shape, lambda g: (0, 0)),
            pl.BlockSpec(att.shape, lambda g: (0, 0)),
            pl.BlockSpec(bias2d.shape, lambda g: (0, 0)),
            pl.BlockSpec(W_cls.shape, lambda g: (0, 0)),
            pl.BlockSpec(bcls2d.shape, lambda g: (0, 0)),
        ],
        out_specs=pl.BlockSpec((N, NUM_AU), lambda g: (0, 0)),
        out_shape=jax.ShapeDtypeStruct((N, NUM_AU), jnp.float32),
    )(pooled16, eiT, W_lin, att, bias2d, W_cls, bcls2d)
    return out


# speedup vs baseline: 3.1947x; 3.1947x over previous
import jax
import jax.numpy as jnp
from jax import lax
from jax.experimental import pallas as pl
from jax.experimental.pallas import tpu as pltpu

N = 32
IN_CH = 512
HIDDEN = 256
HEADS = 4
OUT_CH = HIDDEN // HEADS
SPATIAL = 56 * 56
NUM_AU = 32
K_STREAMS = 4
CH_BLK = IN_CH // K_STREAMS


def _fused_body(*refs):
    x_refs = refs[:K_STREAMS]
    (eiT_ref, wlin_ref, att_ref, bias_ref, wcls_ref, bcls_ref, out_ref,
     acc_ref) = refs[K_STREAMS:]
    g = pl.program_id(0)
    num_steps = pl.num_programs(0)

    col = lax.broadcasted_iota(jnp.int32, (56, N), 1)
    sel = (col == g).astype(jnp.float32)
    for k in range(K_STREAMS):
        r = jnp.sum(x_refs[k][...], axis=1)
        s_wide = jnp.dot(r, sel,
                         preferred_element_type=jnp.float32)

        @pl.when(g == 0)
        def _init(s_wide=s_wide, k=k):
            acc_ref[pl.ds(k * CH_BLK, CH_BLK), :] = s_wide

        @pl.when(g > 0)
        def _accum(s_wide=s_wide, k=k):
            acc_ref[pl.ds(k * CH_BLK, CH_BLK), :] += s_wide

    @pl.when(g == num_steps - 1)
    def _epilogue():
        xT = acc_ref[...] * (1.0 / SPATIAL)
        h = lax.dot_general(xT, wlin_ref[...], (((0,), (0,)), ((), ())),
                            preferred_element_type=jnp.float32)
        hh = h.reshape(N, HEADS, OUT_CH)
        att_src = att_ref[0:HEADS, :]
        att_dst = att_ref[HEADS:2 * HEADS, :]
        a_src = jnp.sum(hh * att_src[None, :, :], axis=-1)
        a_dst = jnp.sum(hh * att_dst[None, :, :], axis=-1)

        e = a_src[:, None, :] + a_dst[None, :, :]
        e = jnp.where(e >= 0, e, 0.2 * e)

        src = eiT_ref[:, 0:1]
        dst = eiT_ref[:, 1:2]
        num_edges = eiT_ref.shape[0]
        ids = lax.broadcasted_iota(jnp.int32, (num_edges, N), 1)
        oh_s = (src == ids).astype(jnp.float32)
        oh_d = (dst == ids).astype(jnp.float32)
        cmat = lax.dot_general(oh_s, oh_d, (((0,), (0,)), ((), ())),
                               preferred_element_type=jnp.float32)
        present = (cmat > 0.0).astype(jnp.float32)

        e_m = e + (present[:, :, None] - 1.0) * jnp.float32(1e30)
        mx = jnp.max(e_m, axis=0)
        ee = jnp.exp(jnp.minimum(e - mx[None, :, :], 0.0))
        ee = ee * cmat[:, :, None]
        denom = jnp.sum(ee, axis=0) + jnp.float32(1e-16)

        outs = []
        for hd in range(HEADS):
            w = ee[:, :, hd]
            num = lax.dot_general(w, h[:, hd * OUT_CH:(hd + 1) * OUT_CH],
                                  (((0,), (0,)), ((), ())),
                                  preferred_element_type=jnp.float32)
            outs.append(num / denom[:, hd][:, None])
        gat = jnp.concatenate(outs, axis=1) + bias_ref[...]
        act = gat * jax.nn.sigmoid(gat)
        logit = jnp.dot(act, wcls_ref[...],
                        preferred_element_type=jnp.float32) + bcls_ref[...]
        out_ref[...] = logit


def kernel(roi_feats, edge_index, W_lin, att_src, att_dst, bias_gat,
           W_cls, b_cls):
    x3d = roi_feats.reshape(N * IN_CH, 56, 56)
    eiT = edge_index.T
    att = jnp.concatenate([att_src, att_dst], axis=0)
    bias2d = bias_gat.reshape(1, HIDDEN)
    bcls2d = b_cls.reshape(1, NUM_AU)
    grid = (N,)

    stream_specs = [
        pl.BlockSpec((CH_BLK, 56, 56),
                     lambda g, k=k: (g * K_STREAMS + k, 0, 0))
        for k in range(K_STREAMS)
    ]
    out = pl.pallas_call(
        _fused_body,
        grid=grid,
        in_specs=stream_specs + [
            pl.BlockSpec(eiT.shape, lambda g: (0, 0)),
            pl.BlockSpec(W_lin.shape, lambda g: (0, 0)),
            pl.BlockSpec(att.shape, lambda g: (0, 0)),
            pl.BlockSpec(bias2d.shape, lambda g: (0, 0)),
            pl.BlockSpec(W_cls.shape, lambda g: (0, 0)),
            pl.BlockSpec(bcls2d.shape, lambda g: (0, 0)),
        ],
        out_specs=pl.BlockSpec((N, NUM_AU), lambda g: (0, 0)),
        out_shape=jax.ShapeDtypeStruct((N, NUM_AU), jnp.float32),
        scratch_shapes=[pltpu.VMEM((IN_CH, N), jnp.float32)],
        compiler_params=pltpu.CompilerParams(
            dimension_semantics=("arbitrary",)),
    )(*([x3d] * K_STREAMS), eiT, W_lin, att, bias2d, W_cls, bcls2d)
    return out
